# baseline (speedup 1.0000x reference)
import jax
import jax.numpy as jnp
from jax import lax
from jax.experimental import pallas as pl

_B = 4096
_NV = 7
_NM = 3
_NVM = _NV * _NM
_R = 2187
_TB = 1024


def _onehot(rows, cols, row_of_col):
    h = lax.broadcasted_iota(jnp.int32, (rows, cols), 0)
    return (h == row_of_col).astype(jnp.float32)


def _body(x_ref, out_ref):
    x = x_ref[...]
    x0, x1, x2 = x[:, 0:1], x[:, 1:2], x[:, 2:3]
    c9 = lax.broadcasted_iota(jnp.int32, (1, 9), 1)

    def pair(va, vb):
        sa = jnp.where(
            c9 // 3 == 0,
            x[:, va : va + 1],
            jnp.where(c9 // 3 == 1, x[:, va + 1 : va + 2], x[:, va + 2 : va + 3]),
        )
        sb = jnp.where(
            c9 % 3 == 0,
            x[:, vb : vb + 1],
            jnp.where(c9 % 3 == 1, x[:, vb + 1 : vb + 2], x[:, vb + 2 : vb + 3]),
        )
        return sa * sb

    h12 = pair(0, 3)
    h34 = pair(6, 9)
    h56 = pair(12, 15)

    u0 = _onehot(9, 81, lax.broadcasted_iota(jnp.int32, (9, 81), 1) // 9)
    u1 = _onehot(9, 81, lax.broadcasted_iota(jnp.int32, (9, 81), 1) % 9)
    h1 = jnp.dot(h12, u0, preferred_element_type=jnp.float32) * jnp.dot(
        h34, u1, preferred_element_type=jnp.float32
    )

    v0 = _onehot(9, 27, lax.broadcasted_iota(jnp.int32, (9, 27), 1) // 3)
    v1 = _onehot(3, 27, lax.broadcasted_iota(jnp.int32, (3, 27), 1) % 3)
    x6 = x[:, 18:21]
    h2 = jnp.dot(h56, v0, preferred_element_type=jnp.float32) * jnp.dot(
        x6, v1, preferred_element_type=jnp.float32
    )

    p1 = _onehot(81, _R, lax.broadcasted_iota(jnp.int32, (81, _R), 1) // 27)
    p2 = _onehot(27, _R, lax.broadcasted_iota(jnp.int32, (27, _R), 1) % 27)
    out_ref[...] = jnp.dot(h1, p1, preferred_element_type=jnp.float32) * jnp.dot(
        h2, p2, preferred_element_type=jnp.float32
    )


def kernel(x, mf_indices):
    del mf_indices
    xf = x.reshape(_B, _NVM)
    grid = (_B // _TB,)
    return pl.pallas_call(
        _body,
        grid=grid,
        in_specs=[pl.BlockSpec((_TB, _NVM), lambda i: (i, 0))],
        out_specs=pl.BlockSpec((_TB, _R), lambda i: (i, 0)),
        out_shape=jax.ShapeDtypeStruct((_B, _R), jnp.float32),
    )(xf)


# speedup vs baseline: 1.2151x; 1.2151x over previous
import jax
import jax.numpy as jnp
from jax import lax
from jax.experimental import pallas as pl

_B = 4096
_NV = 7
_NM = 3
_NVM = _NV * _NM
_R = 2187
_TB = 512


def _body(x_ref, idx_ref, out_ref):
    lx = jnp.log2(x_ref[...])
    idx7 = idx_ref[0:_NV, :]
    idx21 = jnp.repeat(idx7, _NM, axis=0)
    which = lax.broadcasted_iota(jnp.int32, (_NVM, _R), 0) % _NM
    m = (idx21 == which).astype(jnp.float32)
    s = jnp.dot(lx, m, preferred_element_type=jnp.float32)
    out_ref[...] = jnp.exp2(s)


def kernel(x, mf_indices):
    xf = x.reshape(_B, _NVM)
    idx_t = jnp.pad(mf_indices.T, ((0, 1), (0, 0)))
    grid = (_B // _TB,)
    return pl.pallas_call(
        _body,
        grid=grid,
        in_specs=[
            pl.BlockSpec((_TB, _NVM), lambda i: (i, 0)),
            pl.BlockSpec((8, _R), lambda i: (0, 0)),
        ],
        out_specs=pl.BlockSpec((_TB, _R), lambda i: (i, 0)),
        out_shape=jax.ShapeDtypeStruct((_B, _R), jnp.float32),
    )(xf, idx_t)
